# baseline (speedup 1.0000x reference)
import functools

import jax
import jax.numpy as jnp
from jax import lax
from jax.experimental import pallas as pl
from jax.experimental.pallas import tpu as pltpu
from jax.experimental.pallas import tpu_sc as plsc

N = 10000
NPAD = 10240
D = 128
E = 320000
NC = 2
NS = 16
NW = NC * NS
EPW = E // NW
RPT = NPAD // NS
ZR = 40

CH_D = 80
NCHUNK_D = EPW // CH_D
NBUF_D = 5

CH = 16
NCHUNK_A = EPW // CH
GS = 5
NB = 2 * GS
LASTG = NCHUNK_A // GS - 1

BT = 1024
BO = 1000
GT = NPAD // BT

f32 = jnp.float32

_mesh = plsc.VectorSubcoreMesh(core_axis_name="c", subcore_axis_name="s")


@functools.partial(
    pl.kernel,
    out_type=jax.ShapeDtypeStruct((NC, NPAD), f32),
    mesh=_mesh,
    scratch_types=[
        pltpu.VMEM((NCHUNK_D, 1, CH_D), jnp.int32),
        pltpu.VMEM((CH_D,), f32),
        pltpu.VMEM((RPT,), f32),
        pltpu.VMEM_SHARED((NPAD,), f32),
    ] + [pltpu.SemaphoreType.DMA] * NBUF_D,
)
def _deg_kernel(dst_hbm, cnt_out, didx, ones_v, zrow, acc, *sems):
    cid = lax.axis_index("c")
    sid = lax.axis_index("s")
    wid = cid * NS + sid
    for j in range(CH_D // 16):
        ones_v[pl.ds(j * 16, 16)] = jnp.ones((16,), f32)
    for j in range(RPT // 16):
        zrow[pl.ds(j * 16, 16)] = jnp.zeros((16,), f32)
    r0 = sid * RPT
    pltpu.sync_copy(dst_hbm.at[pl.ds(wid * NCHUNK_D, NCHUNK_D)], didx)
    pltpu.sync_copy(zrow, acc.at[pl.ds(r0, RPT)])
    plsc.subcore_barrier()

    def fire(c, b):
        pltpu.async_copy(ones_v, acc.at[didx.at[c, 0]], sems[b], add=True)

    def drain(c, b):
        pltpu.make_async_copy(ones_v, acc.at[didx.at[c, 0]], sems[b]).wait()

    for b in range(NBUF_D):
        fire(b, b)

    def body(g, carry):
        for b in range(NBUF_D):
            c = g * NBUF_D + b
            drain(c, b)
            fire(c + NBUF_D, b)
        return carry

    lax.fori_loop(0, NCHUNK_D // NBUF_D - 1, body, 0)
    for b in range(NBUF_D):
        drain((NCHUNK_D // NBUF_D - 1) * NBUF_D + b, b)
    plsc.subcore_barrier()
    pltpu.sync_copy(acc.at[pl.ds(r0, RPT)], cnt_out.at[cid, pl.ds(r0, RPT)])


def _make_agg(with_scalar):
    out_type = [jax.ShapeDtypeStruct((NC, NPAD, D), f32)]
    scratch = [
        pltpu.VMEM((NB, 1, CH), jnp.int32),
        pltpu.VMEM((NB, 1, CH), jnp.int32),
        pltpu.VMEM((NB, CH, D), f32),
        pltpu.VMEM((ZR, D), f32),
        pltpu.VMEM_SHARED((NPAD, D), f32),
    ]
    if with_scalar:
        out_type.append(jax.ShapeDtypeStruct((NC, NPAD), f32))
        scratch += [
            pltpu.VMEM((NB, 1, CH), f32),
            pltpu.VMEM((RPT,), f32),
            pltpu.VMEM_SHARED((NPAD,), f32),
        ]
    scratch += [pltpu.SemaphoreType.DMA] * (3 * NB)

    def body(*args):
        if with_scalar:
            (x_hbm, dvec_hbm, src_hbm, dst_hbm, yout, sout,
             sidx, didx, rows, zblk, yacc, svals, zrow, sacc) = args[:14]
            sems = args[14:]
        else:
            (x_hbm, src_hbm, dst_hbm, yout,
             sidx, didx, rows, zblk, yacc) = args[:9]
            sems = args[9:]
        isem, gsem, rsem = sems[:NB], sems[NB:2 * NB], sems[2 * NB:3 * NB]
        cid = lax.axis_index("c")
        sid = lax.axis_index("s")
        wid = cid * NS + sid
        r0 = sid * RPT
        base = wid * EPW

        def fire_idx(c, u):
            off = pl.multiple_of(base + c * CH, 8)
            pltpu.async_copy(src_hbm.at[pl.ds(off, CH)], sidx.at[u, 0],
                             isem[u])
            pltpu.async_copy(dst_hbm.at[pl.ds(off, CH)], didx.at[u, 0],
                             isem[u])

        def wait_idx(u):
            off = pl.multiple_of(base, 8)
            pltpu.make_async_copy(src_hbm.at[pl.ds(off, CH)], sidx.at[u, 0],
                                  isem[u]).wait()
            pltpu.make_async_copy(dst_hbm.at[pl.ds(off, CH)], didx.at[u, 0],
                                  isem[u]).wait()

        def fire_gather(u):
            pltpu.async_copy(x_hbm.at[sidx.at[u, 0]], rows.at[u], gsem[u])
            if with_scalar:
                pltpu.async_copy(dvec_hbm.at[sidx.at[u, 0]], svals.at[u, 0],
                                 gsem[u])

        def wait_gather(u):
            pltpu.make_async_copy(x_hbm.at[sidx.at[u, 0]], rows.at[u],
                                  gsem[u]).wait()
            if with_scalar:
                pltpu.make_async_copy(dvec_hbm.at[sidx.at[u, 0]],
                                      svals.at[u, 0], gsem[u]).wait()

        def fire_scatter(u):
            pltpu.async_copy(rows.at[u], yacc.at[didx.at[u, 0]], rsem[u],
                             add=True)
            if with_scalar:
                pltpu.async_copy(svals.at[u, 0], sacc.at[didx.at[u, 0]],
                                 rsem[u], add=True)

        def wait_scatter(u):
            pltpu.make_async_copy(rows.at[u], yacc.at[didx.at[u, 0]],
                                  rsem[u]).wait()
            if with_scalar:
                pltpu.make_async_copy(svals.at[u, 0],
                                      sacc.at[didx.at[u, 0]], rsem[u]).wait()

        def stage(g, par, do_next_gather, do_idx_load):
            s = par * GS
            s1 = (1 - par) * GS
            for b in range(GS):
                wait_gather(s + b)
                fire_scatter(s + b)
            if do_next_gather:
                for b in range(GS):
                    wait_idx(s1 + b)
                    fire_gather(s1 + b)
            for b in range(GS):
                wait_scatter(s + b)
                if do_idx_load:
                    fire_idx((g + 2) * GS + b, s + b)

        for b in range(GS):
            fire_idx(b, b)
        for b in range(GS):
            wait_idx(b)
            fire_gather(b)
        for b in range(GS):
            fire_idx(GS + b, GS + b)

        for i in range(ZR):
            for j in range(D // 16):
                zblk[i, pl.ds(j * 16, 16)] = jnp.zeros((16,), f32)
        for t in range(RPT // ZR):
            pltpu.sync_copy(zblk, yacc.at[pl.ds(r0 + t * ZR, ZR)])
        if with_scalar:
            for j in range(RPT // 16):
                zrow[pl.ds(j * 16, 16)] = jnp.zeros((16,), f32)
            pltpu.sync_copy(zrow, sacc.at[pl.ds(r0, RPT)])
        plsc.subcore_barrier()

        def body_g(k, carry):
            stage(2 * k, 0, True, True)
            stage(2 * k + 1, 1, True, True)
            return carry

        lax.fori_loop(0, (LASTG - 2) // 2, body_g, 0)
        stage(LASTG - 2, (LASTG - 2) % 2, True, True)
        stage(LASTG - 1, (LASTG - 1) % 2, True, False)
        stage(LASTG, LASTG % 2, False, False)
        plsc.subcore_barrier()
        pltpu.sync_copy(yacc.at[pl.ds(r0, RPT)], yout.at[cid, pl.ds(r0, RPT)])
        if with_scalar:
            pltpu.sync_copy(sacc.at[pl.ds(r0, RPT)],
                            sout.at[cid, pl.ds(r0, RPT)])

    return functools.partial(
        pl.kernel,
        out_type=tuple(out_type) if with_scalar else out_type[0],
        mesh=_mesh,
        scratch_types=scratch,
    )(body)


_agg_scalar = _make_agg(True)
_agg_plain = _make_agg(False)


def _prep_body(c0_ref, c1_ref, emb_ref, t1_ref, dis_ref, inv_ref):
    deg = c0_ref[...] + c1_ref[...] + 1.0
    dis = lax.rsqrt(deg)
    dis_ref[...] = dis
    inv_ref[...] = 1.0 / deg
    t1_ref[...] = dis * emb_ref[...]


def _mid_body(p0_ref, p1_ref, t1_ref, inv_ref, dis_ref, sv0_ref, sv1_ref,
              t3_ref, s_ref):
    t2 = p0_ref[...] + p1_ref[...] + t1_ref[...]
    t3_ref[...] = inv_ref[...] * t2
    dis = dis_ref[...]
    s_ref[...] = dis * (sv0_ref[...] + sv1_ref[...] + dis)


def _final_body(q0_ref, q1_ref, t3_ref, dis_ref, s_ref, w1_ref, w2_ref,
                b1_ref, b2_ref, out_ref):
    q = dis_ref[...] * (q0_ref[...] + q1_ref[...] + t3_ref[...])
    h = jnp.dot(q, w1_ref[...], preferred_element_type=f32) \
        + s_ref[...] * b1_ref[...]
    out_ref[...] = jnp.dot(h, w2_ref[...], preferred_element_type=f32) \
                   + b2_ref[...]


def _col_spec():
    return pl.BlockSpec((BT, 1), lambda i: (i, 0))


def _row_spec():
    return pl.BlockSpec((BT, D), lambda i: (i, 0))


def _full_spec(shape):
    return pl.BlockSpec(shape, lambda i: (0, 0))


def kernel(emb, edge_index, W1, b1, W2, b2):
    src1 = edge_index[0]
    dst1 = edge_index[1]
    dst3 = dst1.reshape(E // CH_D, 1, CH_D)
    embp = jnp.pad(emb, ((0, NPAD - N), (0, 0)))

    cnt = _deg_kernel(dst3)

    t1, dis, inv = pl.pallas_call(
        _prep_body,
        grid=(GT,),
        in_specs=[_col_spec(), _col_spec(), _row_spec()],
        out_specs=[_row_spec(), _col_spec(), _col_spec()],
        out_shape=(jax.ShapeDtypeStruct((NPAD, D), f32),
                   jax.ShapeDtypeStruct((NPAD, 1), f32),
                   jax.ShapeDtypeStruct((NPAD, 1), f32)),
    )(cnt[0].reshape(NPAD, 1), cnt[1].reshape(NPAD, 1), embp)

    t2p, svp = _agg_scalar(t1, dis.reshape(NPAD), src1, dst1)

    t3, svec = pl.pallas_call(
        _mid_body,
        grid=(GT,),
        in_specs=[_row_spec(), _row_spec(), _row_spec(), _col_spec(),
                  _col_spec(), _col_spec(), _col_spec()],
        out_specs=[_row_spec(), _col_spec()],
        out_shape=(jax.ShapeDtypeStruct((NPAD, D), f32),
                   jax.ShapeDtypeStruct((NPAD, 1), f32)),
    )(t2p[0], t2p[1], t1, inv, dis,
      svp[0].reshape(NPAD, 1), svp[1].reshape(NPAD, 1))

    t4p = _agg_plain(t3, src1, dst1)

    bo = pl.BlockSpec((BO, D), lambda i: (i, 0))
    bo1 = pl.BlockSpec((BO, 1), lambda i: (i, 0))
    out = pl.pallas_call(
        _final_body,
        grid=(N // BO,),
        in_specs=[bo, bo, bo, bo1, bo1, _full_spec((D, 2 * D)),
                  _full_spec((2 * D, D)), _full_spec((1, 2 * D)),
                  _full_spec((1, D))],
        out_specs=bo,
        out_shape=jax.ShapeDtypeStruct((N, D), f32),
    )(t4p[0], t4p[1], t3, dis, svec, W1, W2, b1.reshape(1, -1),
      b2.reshape(1, -1))

    return out


# speedup vs baseline: 1.0854x; 1.0854x over previous
import functools

import jax
import jax.numpy as jnp
from jax import lax
from jax.experimental import pallas as pl
from jax.experimental.pallas import tpu as pltpu
from jax.experimental.pallas import tpu_sc as plsc

N = 10000
NPAD = 10240
D = 128
E = 320000
NC = 2
NS = 16
NW = NC * NS
EPW = E // NW
RPT = NPAD // NS
ZR = 40

CH_D = 80
NCHUNK_D = EPW // CH_D
NBUF_D = 5

CH = 80
NCHUNK_A = EPW // CH
GS = 1
NB = 2 * GS
LASTG = NCHUNK_A // GS - 1

BT = 1024
BO = 1000
GT = NPAD // BT

f32 = jnp.float32

_mesh = plsc.VectorSubcoreMesh(core_axis_name="c", subcore_axis_name="s")


@functools.partial(
    pl.kernel,
    out_type=jax.ShapeDtypeStruct((NC, NPAD), f32),
    mesh=_mesh,
    scratch_types=[
        pltpu.VMEM((NCHUNK_D, 1, CH_D), jnp.int32),
        pltpu.VMEM((CH_D,), f32),
        pltpu.VMEM((RPT,), f32),
        pltpu.VMEM_SHARED((NPAD,), f32),
    ] + [pltpu.SemaphoreType.DMA] * NBUF_D,
)
def _deg_kernel(dst_hbm, cnt_out, didx, ones_v, zrow, acc, *sems):
    cid = lax.axis_index("c")
    sid = lax.axis_index("s")
    wid = cid * NS + sid
    for j in range(CH_D // 16):
        ones_v[pl.ds(j * 16, 16)] = jnp.ones((16,), f32)
    for j in range(RPT // 16):
        zrow[pl.ds(j * 16, 16)] = jnp.zeros((16,), f32)
    r0 = sid * RPT
    pltpu.sync_copy(dst_hbm.at[pl.ds(wid * NCHUNK_D, NCHUNK_D)], didx)
    pltpu.sync_copy(zrow, acc.at[pl.ds(r0, RPT)])
    plsc.subcore_barrier()

    def fire(c, b):
        pltpu.async_copy(ones_v, acc.at[didx.at[c, 0]], sems[b], add=True)

    def drain(c, b):
        pltpu.make_async_copy(ones_v, acc.at[didx.at[c, 0]], sems[b]).wait()

    for b in range(NBUF_D):
        fire(b, b)

    def body(g, carry):
        for b in range(NBUF_D):
            c = g * NBUF_D + b
            drain(c, b)
            fire(c + NBUF_D, b)
        return carry

    lax.fori_loop(0, NCHUNK_D // NBUF_D - 1, body, 0)
    for b in range(NBUF_D):
        drain((NCHUNK_D // NBUF_D - 1) * NBUF_D + b, b)
    plsc.subcore_barrier()
    pltpu.sync_copy(acc.at[pl.ds(r0, RPT)], cnt_out.at[cid, pl.ds(r0, RPT)])


def _make_agg(with_scalar):
    out_type = [jax.ShapeDtypeStruct((NC, NPAD, D), f32)]
    scratch = [
        pltpu.VMEM((NB, 1, CH), jnp.int32),
        pltpu.VMEM((NB, 1, CH), jnp.int32),
        pltpu.VMEM((NB, CH, D), f32),
        pltpu.VMEM((ZR, D), f32),
        pltpu.VMEM_SHARED((NPAD, D), f32),
    ]
    if with_scalar:
        out_type.append(jax.ShapeDtypeStruct((NC, NPAD), f32))
        scratch += [
            pltpu.VMEM((NB, 1, CH), f32),
            pltpu.VMEM((RPT,), f32),
            pltpu.VMEM_SHARED((NPAD,), f32),
        ]
    scratch += [pltpu.SemaphoreType.DMA] * (3 * NB)

    def body(*args):
        if with_scalar:
            (x_hbm, dvec_hbm, src_hbm, dst_hbm, yout, sout,
             sidx, didx, rows, zblk, yacc, svals, zrow, sacc) = args[:14]
            sems = args[14:]
        else:
            (x_hbm, src_hbm, dst_hbm, yout,
             sidx, didx, rows, zblk, yacc) = args[:9]
            sems = args[9:]
        isem, gsem, rsem = sems[:NB], sems[NB:2 * NB], sems[2 * NB:3 * NB]
        cid = lax.axis_index("c")
        sid = lax.axis_index("s")
        wid = cid * NS + sid
        r0 = sid * RPT
        base = wid * EPW

        def fire_idx(c, u):
            off = pl.multiple_of(base + c * CH, 8)
            pltpu.async_copy(src_hbm.at[pl.ds(off, CH)], sidx.at[u, 0],
                             isem[u])
            pltpu.async_copy(dst_hbm.at[pl.ds(off, CH)], didx.at[u, 0],
                             isem[u])

        def wait_idx(u):
            off = pl.multiple_of(base, 8)
            pltpu.make_async_copy(src_hbm.at[pl.ds(off, CH)], sidx.at[u, 0],
                                  isem[u]).wait()
            pltpu.make_async_copy(dst_hbm.at[pl.ds(off, CH)], didx.at[u, 0],
                                  isem[u]).wait()

        def fire_gather(u):
            pltpu.async_copy(x_hbm.at[sidx.at[u, 0]], rows.at[u], gsem[u])
            if with_scalar:
                pltpu.async_copy(dvec_hbm.at[sidx.at[u, 0]], svals.at[u, 0],
                                 gsem[u])

        def wait_gather(u):
            pltpu.make_async_copy(x_hbm.at[sidx.at[u, 0]], rows.at[u],
                                  gsem[u]).wait()
            if with_scalar:
                pltpu.make_async_copy(dvec_hbm.at[sidx.at[u, 0]],
                                      svals.at[u, 0], gsem[u]).wait()

        def fire_scatter(u):
            pltpu.async_copy(rows.at[u], yacc.at[didx.at[u, 0]], rsem[u],
                             add=True)
            if with_scalar:
                pltpu.async_copy(svals.at[u, 0], sacc.at[didx.at[u, 0]],
                                 rsem[u], add=True)

        def wait_scatter(u):
            pltpu.make_async_copy(rows.at[u], yacc.at[didx.at[u, 0]],
                                  rsem[u]).wait()
            if with_scalar:
                pltpu.make_async_copy(svals.at[u, 0],
                                      sacc.at[didx.at[u, 0]], rsem[u]).wait()

        def stage(g, par, do_next_gather, do_idx_load):
            s = par * GS
            s1 = (1 - par) * GS
            for b in range(GS):
                wait_gather(s + b)
                fire_scatter(s + b)
            if do_next_gather:
                for b in range(GS):
                    wait_idx(s1 + b)
                    fire_gather(s1 + b)
            for b in range(GS):
                wait_scatter(s + b)
                if do_idx_load:
                    fire_idx((g + 2) * GS + b, s + b)

        for b in range(GS):
            fire_idx(b, b)
        for b in range(GS):
            wait_idx(b)
            fire_gather(b)
        for b in range(GS):
            fire_idx(GS + b, GS + b)

        for i in range(ZR):
            for j in range(D // 16):
                zblk[i, pl.ds(j * 16, 16)] = jnp.zeros((16,), f32)
        for t in range(RPT // ZR):
            pltpu.sync_copy(zblk, yacc.at[pl.ds(r0 + t * ZR, ZR)])
        if with_scalar:
            for j in range(RPT // 16):
                zrow[pl.ds(j * 16, 16)] = jnp.zeros((16,), f32)
            pltpu.sync_copy(zrow, sacc.at[pl.ds(r0, RPT)])
        plsc.subcore_barrier()

        def body_g(k, carry):
            stage(2 * k, 0, True, True)
            stage(2 * k + 1, 1, True, True)
            return carry

        lax.fori_loop(0, (LASTG - 2) // 2, body_g, 0)
        stage(LASTG - 2, (LASTG - 2) % 2, True, True)
        stage(LASTG - 1, (LASTG - 1) % 2, True, False)
        stage(LASTG, LASTG % 2, False, False)
        plsc.subcore_barrier()
        pltpu.sync_copy(yacc.at[pl.ds(r0, RPT)], yout.at[cid, pl.ds(r0, RPT)])
        if with_scalar:
            pltpu.sync_copy(sacc.at[pl.ds(r0, RPT)],
                            sout.at[cid, pl.ds(r0, RPT)])

    return functools.partial(
        pl.kernel,
        out_type=tuple(out_type) if with_scalar else out_type[0],
        mesh=_mesh,
        scratch_types=scratch,
    )(body)


_agg_scalar = _make_agg(True)
_agg_plain = _make_agg(False)


def _prep_body(c0_ref, c1_ref, emb_ref, t1_ref, dis_ref, inv_ref):
    deg = c0_ref[...] + c1_ref[...] + 1.0
    dis = lax.rsqrt(deg)
    dis_ref[...] = dis
    inv_ref[...] = 1.0 / deg
    t1_ref[...] = dis * emb_ref[...]


def _mid_body(p0_ref, p1_ref, t1_ref, inv_ref, dis_ref, sv0_ref, sv1_ref,
              t3_ref, s_ref):
    t2 = p0_ref[...] + p1_ref[...] + t1_ref[...]
    t3_ref[...] = inv_ref[...] * t2
    dis = dis_ref[...]
    s_ref[...] = dis * (sv0_ref[...] + sv1_ref[...] + dis)


def _final_body(q0_ref, q1_ref, t3_ref, dis_ref, s_ref, w1_ref, w2_ref,
                b1_ref, b2_ref, out_ref):
    q = dis_ref[...] * (q0_ref[...] + q1_ref[...] + t3_ref[...])
    h = jnp.dot(q, w1_ref[...], preferred_element_type=f32) \
        + s_ref[...] * b1_ref[...]
    out_ref[...] = jnp.dot(h, w2_ref[...], preferred_element_type=f32) \
                   + b2_ref[...]


def _col_spec():
    return pl.BlockSpec((BT, 1), lambda i: (i, 0))


def _row_spec():
    return pl.BlockSpec((BT, D), lambda i: (i, 0))


def _full_spec(shape):
    return pl.BlockSpec(shape, lambda i: (0, 0))


def kernel(emb, edge_index, W1, b1, W2, b2):
    src1 = edge_index[0]
    dst1 = edge_index[1]
    dst3 = dst1.reshape(E // CH_D, 1, CH_D)
    embp = jnp.pad(emb, ((0, NPAD - N), (0, 0)))

    cnt = _deg_kernel(dst3)

    t1, dis, inv = pl.pallas_call(
        _prep_body,
        grid=(GT,),
        in_specs=[_col_spec(), _col_spec(), _row_spec()],
        out_specs=[_row_spec(), _col_spec(), _col_spec()],
        out_shape=(jax.ShapeDtypeStruct((NPAD, D), f32),
                   jax.ShapeDtypeStruct((NPAD, 1), f32),
                   jax.ShapeDtypeStruct((NPAD, 1), f32)),
    )(cnt[0].reshape(NPAD, 1), cnt[1].reshape(NPAD, 1), embp)

    t2p, svp = _agg_scalar(t1, dis.reshape(NPAD), src1, dst1)

    t3, svec = pl.pallas_call(
        _mid_body,
        grid=(GT,),
        in_specs=[_row_spec(), _row_spec(), _row_spec(), _col_spec(),
                  _col_spec(), _col_spec(), _col_spec()],
        out_specs=[_row_spec(), _col_spec()],
        out_shape=(jax.ShapeDtypeStruct((NPAD, D), f32),
                   jax.ShapeDtypeStruct((NPAD, 1), f32)),
    )(t2p[0], t2p[1], t1, inv, dis,
      svp[0].reshape(NPAD, 1), svp[1].reshape(NPAD, 1))

    t4p = _agg_plain(t3, src1, dst1)

    bo = pl.BlockSpec((BO, D), lambda i: (i, 0))
    bo1 = pl.BlockSpec((BO, 1), lambda i: (i, 0))
    out = pl.pallas_call(
        _final_body,
        grid=(N // BO,),
        in_specs=[bo, bo, bo, bo1, bo1, _full_spec((D, 2 * D)),
                  _full_spec((2 * D, D)), _full_spec((1, 2 * D)),
                  _full_spec((1, D))],
        out_specs=bo,
        out_shape=jax.ShapeDtypeStruct((N, D), f32),
    )(t4p[0], t4p[1], t3, dis, svec, W1, W2, b1.reshape(1, -1),
      b2.reshape(1, -1))

    return out


# speedup vs baseline: 1.2137x; 1.1182x over previous
import functools

import jax
import jax.numpy as jnp
from jax import lax
from jax.experimental import pallas as pl
from jax.experimental.pallas import tpu as pltpu
from jax.experimental.pallas import tpu_sc as plsc

N = 10000
NPAD = 10240
D = 128
E = 320000
NC = 2
NS = 16
NW = NC * NS
EPW = E // NW
RPT = NPAD // NS
ZR = 40

CH_D = 80
NCHUNK_D = EPW // CH_D
NBUF_D = 5

CH = 40
NCHUNK_A = EPW // CH
GS = 2
NB = 2 * GS
LASTG = NCHUNK_A // GS - 1

BT = 1024
BO = 1000
GT = NPAD // BT

f32 = jnp.float32

_mesh = plsc.VectorSubcoreMesh(core_axis_name="c", subcore_axis_name="s")


@functools.partial(
    pl.kernel,
    out_type=jax.ShapeDtypeStruct((NC, NPAD), f32),
    mesh=_mesh,
    scratch_types=[
        pltpu.VMEM((NCHUNK_D, 1, CH_D), jnp.int32),
        pltpu.VMEM((CH_D,), f32),
        pltpu.VMEM((RPT,), f32),
        pltpu.VMEM_SHARED((NPAD,), f32),
    ] + [pltpu.SemaphoreType.DMA] * NBUF_D,
)
def _deg_kernel(dst_hbm, cnt_out, didx, ones_v, zrow, acc, *sems):
    cid = lax.axis_index("c")
    sid = lax.axis_index("s")
    wid = cid * NS + sid
    for j in range(CH_D // 16):
        ones_v[pl.ds(j * 16, 16)] = jnp.ones((16,), f32)
    for j in range(RPT // 16):
        zrow[pl.ds(j * 16, 16)] = jnp.zeros((16,), f32)
    r0 = sid * RPT
    pltpu.sync_copy(dst_hbm.at[pl.ds(wid * NCHUNK_D, NCHUNK_D)], didx)
    pltpu.sync_copy(zrow, acc.at[pl.ds(r0, RPT)])
    plsc.subcore_barrier()

    def fire(c, b):
        pltpu.async_copy(ones_v, acc.at[didx.at[c, 0]], sems[b], add=True)

    def drain(c, b):
        pltpu.make_async_copy(ones_v, acc.at[didx.at[c, 0]], sems[b]).wait()

    for b in range(NBUF_D):
        fire(b, b)

    def body(g, carry):
        for b in range(NBUF_D):
            c = g * NBUF_D + b
            drain(c, b)
            fire(c + NBUF_D, b)
        return carry

    lax.fori_loop(0, NCHUNK_D // NBUF_D - 1, body, 0)
    for b in range(NBUF_D):
        drain((NCHUNK_D // NBUF_D - 1) * NBUF_D + b, b)
    plsc.subcore_barrier()
    pltpu.sync_copy(acc.at[pl.ds(r0, RPT)], cnt_out.at[cid, pl.ds(r0, RPT)])


def _make_agg(with_scalar):
    out_type = [jax.ShapeDtypeStruct((NC, NPAD, D), f32)]
    scratch = [
        pltpu.VMEM((NB, 1, CH), jnp.int32),
        pltpu.VMEM((NB, 1, CH), jnp.int32),
        pltpu.VMEM((NB, CH, D), f32),
        pltpu.VMEM((ZR, D), f32),
        pltpu.VMEM_SHARED((NPAD, D), f32),
    ]
    if with_scalar:
        out_type.append(jax.ShapeDtypeStruct((NC, NPAD), f32))
        scratch += [
            pltpu.VMEM((NB, 1, CH), f32),
            pltpu.VMEM((RPT,), f32),
            pltpu.VMEM_SHARED((NPAD,), f32),
        ]
    scratch += [pltpu.SemaphoreType.DMA] * (3 * NB)

    def body(*args):
        if with_scalar:
            (x_hbm, dvec_hbm, src_hbm, dst_hbm, yout, sout,
             sidx, didx, rows, zblk, yacc, svals, zrow, sacc) = args[:14]
            sems = args[14:]
        else:
            (x_hbm, src_hbm, dst_hbm, yout,
             sidx, didx, rows, zblk, yacc) = args[:9]
            sems = args[9:]
        isem, gsem, rsem = sems[:NB], sems[NB:2 * NB], sems[2 * NB:3 * NB]
        cid = lax.axis_index("c")
        sid = lax.axis_index("s")
        wid = cid * NS + sid
        r0 = sid * RPT
        base = wid * EPW

        def fire_idx(c, u):
            off = pl.multiple_of(base + c * CH, 8)
            pltpu.async_copy(src_hbm.at[pl.ds(off, CH)], sidx.at[u, 0],
                             isem[u])
            pltpu.async_copy(dst_hbm.at[pl.ds(off, CH)], didx.at[u, 0],
                             isem[u])

        def wait_idx(u):
            off = pl.multiple_of(base, 8)
            pltpu.make_async_copy(src_hbm.at[pl.ds(off, CH)], sidx.at[u, 0],
                                  isem[u]).wait()
            pltpu.make_async_copy(dst_hbm.at[pl.ds(off, CH)], didx.at[u, 0],
                                  isem[u]).wait()

        def fire_gather(u):
            pltpu.async_copy(x_hbm.at[sidx.at[u, 0]], rows.at[u], gsem[u])
            if with_scalar:
                pltpu.async_copy(dvec_hbm.at[sidx.at[u, 0]], svals.at[u, 0],
                                 gsem[u])

        def wait_gather(u):
            pltpu.make_async_copy(x_hbm.at[sidx.at[u, 0]], rows.at[u],
                                  gsem[u]).wait()
            if with_scalar:
                pltpu.make_async_copy(dvec_hbm.at[sidx.at[u, 0]],
                                      svals.at[u, 0], gsem[u]).wait()

        def fire_scatter(u):
            pltpu.async_copy(rows.at[u], yacc.at[didx.at[u, 0]], rsem[u],
                             add=True)
            if with_scalar:
                pltpu.async_copy(svals.at[u, 0], sacc.at[didx.at[u, 0]],
                                 rsem[u], add=True)

        def wait_scatter(u):
            pltpu.make_async_copy(rows.at[u], yacc.at[didx.at[u, 0]],
                                  rsem[u]).wait()
            if with_scalar:
                pltpu.make_async_copy(svals.at[u, 0],
                                      sacc.at[didx.at[u, 0]], rsem[u]).wait()

        def stage(g, par, do_next_gather, do_idx_load):
            s = par * GS
            s1 = (1 - par) * GS
            for b in range(GS):
                wait_gather(s + b)
                fire_scatter(s + b)
            if do_next_gather:
                for b in range(GS):
                    wait_idx(s1 + b)
                    fire_gather(s1 + b)
            for b in range(GS):
                wait_scatter(s + b)
                if do_idx_load:
                    fire_idx((g + 2) * GS + b, s + b)

        for b in range(GS):
            fire_idx(b, b)
        for b in range(GS):
            wait_idx(b)
            fire_gather(b)
        for b in range(GS):
            fire_idx(GS + b, GS + b)

        for i in range(ZR):
            for j in range(D // 16):
                zblk[i, pl.ds(j * 16, 16)] = jnp.zeros((16,), f32)
        for t in range(RPT // ZR):
            pltpu.sync_copy(zblk, yacc.at[pl.ds(r0 + t * ZR, ZR)])
        if with_scalar:
            for j in range(RPT // 16):
                zrow[pl.ds(j * 16, 16)] = jnp.zeros((16,), f32)
            pltpu.sync_copy(zrow, sacc.at[pl.ds(r0, RPT)])
        plsc.subcore_barrier()

        def body_g(k, carry):
            stage(2 * k, 0, True, True)
            stage(2 * k + 1, 1, True, True)
            return carry

        lax.fori_loop(0, (LASTG - 2) // 2, body_g, 0)
        stage(LASTG - 2, (LASTG - 2) % 2, True, True)
        stage(LASTG - 1, (LASTG - 1) % 2, True, False)
        stage(LASTG, LASTG % 2, False, False)
        plsc.subcore_barrier()
        pltpu.sync_copy(yacc.at[pl.ds(r0, RPT)], yout.at[cid, pl.ds(r0, RPT)])
        if with_scalar:
            pltpu.sync_copy(sacc.at[pl.ds(r0, RPT)],
                            sout.at[cid, pl.ds(r0, RPT)])

    return functools.partial(
        pl.kernel,
        out_type=tuple(out_type) if with_scalar else out_type[0],
        mesh=_mesh,
        scratch_types=scratch,
    )(body)


_agg_scalar = _make_agg(True)
_agg_plain = _make_agg(False)


def _unpack_col(d):
    pr = lax.broadcasted_iota(jnp.int32, (BT, BT // 128), 0) // 128
    pc = lax.broadcasted_iota(jnp.int32, (BT, BT // 128), 1)
    sel = (pr == pc).astype(f32)
    rows = jnp.dot(sel, d, preferred_element_type=f32)
    lr = lax.broadcasted_iota(jnp.int32, (BT, 128), 0) & 127
    lc = lax.broadcasted_iota(jnp.int32, (BT, 128), 1)
    msk = (lr == lc).astype(f32)
    return jnp.sum(rows * msk, axis=1, keepdims=True)


def _prep_body(c0_ref, c1_ref, emb_ref, t1_ref, disp_ref):
    deg = c0_ref[...] + c1_ref[...] + 1.0
    dis = lax.rsqrt(deg)
    disp_ref[...] = dis
    t1_ref[...] = _unpack_col(dis) * emb_ref[...]


def _mid_body(p0_ref, p1_ref, t1_ref, disp_ref, sv0_ref, sv1_ref,
              t3_ref, svecp_ref):
    dis = disp_ref[...]
    disr = _unpack_col(dis)
    t3_ref[...] = (disr * disr) * (p0_ref[...] + p1_ref[...] + t1_ref[...])
    svecp_ref[...] = dis * (sv0_ref[...] + sv1_ref[...] + dis)


def _final_body(q0_ref, q1_ref, t3_ref, disp_ref, svecp_ref, w1_ref, w2_ref,
                b1_ref, b2_ref, out_ref):
    disr = _unpack_col(disp_ref[...])
    svr = _unpack_col(svecp_ref[...])
    q = disr * (q0_ref[...] + q1_ref[...] + t3_ref[...])
    h = jnp.dot(q, w1_ref[...], preferred_element_type=f32) + svr * b1_ref[...]
    out_ref[...] = jnp.dot(h, w2_ref[...], preferred_element_type=f32) \
                   + b2_ref[...]


def _pack_spec():
    return pl.BlockSpec((BT // 128, 128), lambda i: (i, 0))


def _row_spec():
    return pl.BlockSpec((BT, D), lambda i: (i, 0))


def _full_spec(shape):
    return pl.BlockSpec(shape, lambda i: (0, 0))


NPK = NPAD // 128


def kernel(emb, edge_index, W1, b1, W2, b2):
    src1 = edge_index[0]
    dst1 = edge_index[1]
    dst3 = dst1.reshape(E // CH_D, 1, CH_D)
    embp = jnp.pad(emb, ((0, NPAD - N), (0, 0)))

    cnt = _deg_kernel(dst3)

    t1, disp = pl.pallas_call(
        _prep_body,
        grid=(GT,),
        in_specs=[_pack_spec(), _pack_spec(), _row_spec()],
        out_specs=[_row_spec(), _pack_spec()],
        out_shape=(jax.ShapeDtypeStruct((NPAD, D), f32),
                   jax.ShapeDtypeStruct((NPK, 128), f32)),
    )(cnt[0].reshape(NPK, 128), cnt[1].reshape(NPK, 128), embp)

    t2p, svp = _agg_scalar(t1, disp.reshape(NPAD), src1, dst1)

    t3, svecp = pl.pallas_call(
        _mid_body,
        grid=(GT,),
        in_specs=[_row_spec(), _row_spec(), _row_spec(), _pack_spec(),
                  _pack_spec(), _pack_spec()],
        out_specs=[_row_spec(), _pack_spec()],
        out_shape=(jax.ShapeDtypeStruct((NPAD, D), f32),
                   jax.ShapeDtypeStruct((NPK, 128), f32)),
    )(t2p[0], t2p[1], t1, disp,
      svp[0].reshape(NPK, 128), svp[1].reshape(NPK, 128))

    t4p = _agg_plain(t3, src1, dst1)

    out = pl.pallas_call(
        _final_body,
        grid=(GT,),
        in_specs=[_row_spec(), _row_spec(), _row_spec(), _pack_spec(),
                  _pack_spec(), _full_spec((D, 2 * D)),
                  _full_spec((2 * D, D)), _full_spec((1, 2 * D)),
                  _full_spec((1, D))],
        out_specs=_row_spec(),
        out_shape=jax.ShapeDtypeStruct((NPAD, D), f32),
    )(t4p[0], t4p[1], t3, disp, svecp, W1, W2, b1.reshape(1, -1),
      b2.reshape(1, -1))

    return out[:N]


# speedup vs baseline: 1.2154x; 1.0014x over previous
import functools

import jax
import jax.numpy as jnp
from jax import lax
from jax.experimental import pallas as pl
from jax.experimental.pallas import tpu as pltpu
from jax.experimental.pallas import tpu_sc as plsc

N = 10000
NPAD = 10240
D = 128
E = 320000
NC = 2
NS = 16
NW = NC * NS
EPW = E // NW
RPT = NPAD // NS
ZR = 40

CH_D = 80
NCHUNK_D = EPW // CH_D
NBUF_D = 5

CH = 40
NCHUNK_A = EPW // CH
GS = 2
NB = 2 * GS
LASTG = NCHUNK_A // GS - 1

BT = 1024
GT = NPAD // BT

f32 = jnp.float32

_mesh = plsc.VectorSubcoreMesh(core_axis_name="c", subcore_axis_name="s")


@functools.partial(
    pl.kernel,
    out_type=jax.ShapeDtypeStruct((NC, NPAD), f32),
    mesh=_mesh,
    scratch_types=[
        pltpu.VMEM((NCHUNK_D, 1, CH_D), jnp.int32),
        pltpu.VMEM((CH_D,), f32),
        pltpu.VMEM((RPT,), f32),
        pltpu.VMEM_SHARED((NPAD,), f32),
    ] + [pltpu.SemaphoreType.DMA] * NBUF_D,
)
def _deg_kernel(dst_hbm, cnt_out, didx, ones_v, zrow, acc, *sems):
    cid = lax.axis_index("c")
    sid = lax.axis_index("s")
    wid = cid * NS + sid
    for j in range(CH_D // 16):
        ones_v[pl.ds(j * 16, 16)] = jnp.ones((16,), f32)
    for j in range(RPT // 16):
        zrow[pl.ds(j * 16, 16)] = jnp.zeros((16,), f32)
    r0 = sid * RPT
    pltpu.sync_copy(dst_hbm.at[pl.ds(wid * NCHUNK_D, NCHUNK_D)], didx)
    pltpu.sync_copy(zrow, acc.at[pl.ds(r0, RPT)])
    plsc.subcore_barrier()

    def fire(c, b):
        pltpu.async_copy(ones_v, acc.at[didx.at[c, 0]], sems[b], add=True)

    def drain(c, b):
        pltpu.make_async_copy(ones_v, acc.at[didx.at[c, 0]], sems[b]).wait()

    for b in range(NBUF_D):
        fire(b, b)

    def body(g, carry):
        for b in range(NBUF_D):
            c = g * NBUF_D + b
            drain(c, b)
            fire(c + NBUF_D, b)
        return carry

    lax.fori_loop(0, NCHUNK_D // NBUF_D - 1, body, 0)
    for b in range(NBUF_D):
        drain((NCHUNK_D // NBUF_D - 1) * NBUF_D + b, b)
    plsc.subcore_barrier()
    pltpu.sync_copy(acc.at[pl.ds(r0, RPT)], cnt_out.at[cid, pl.ds(r0, RPT)])


def _make_agg(with_scalar):
    out_type = [jax.ShapeDtypeStruct((NC, NPAD, D), f32)]
    scratch = [
        pltpu.VMEM((NB, 1, CH), jnp.int32),
        pltpu.VMEM((NB, 1, CH), jnp.int32),
        pltpu.VMEM((NB, CH, D), f32),
        pltpu.VMEM((ZR, D), f32),
        pltpu.VMEM_SHARED((NPAD, D), f32),
    ]
    if with_scalar:
        out_type.append(jax.ShapeDtypeStruct((NC, NPAD), f32))
        scratch += [
            pltpu.VMEM((NB, 1, CH), f32),
            pltpu.VMEM((RPT,), f32),
            pltpu.VMEM_SHARED((NPAD,), f32),
        ]
    scratch += [pltpu.SemaphoreType.DMA] * (3 * NB)

    def body(*args):
        if with_scalar:
            (x_hbm, dvec_hbm, src_hbm, dst_hbm, yout, sout,
             sidx, didx, rows, zblk, yacc, svals, zrow, sacc) = args[:14]
            sems = args[14:]
        else:
            (x_hbm, src_hbm, dst_hbm, yout,
             sidx, didx, rows, zblk, yacc) = args[:9]
            sems = args[9:]
        isem, gsem, rsem = sems[:NB], sems[NB:2 * NB], sems[2 * NB:3 * NB]
        cid = lax.axis_index("c")
        sid = lax.axis_index("s")
        wid = cid * NS + sid
        r0 = sid * RPT
        base = wid * EPW

        def fire_idx(c, u):
            off = pl.multiple_of(base + c * CH, 8)
            pltpu.async_copy(src_hbm.at[pl.ds(off, CH)], sidx.at[u, 0],
                             isem[u])
            pltpu.async_copy(dst_hbm.at[pl.ds(off, CH)], didx.at[u, 0],
                             isem[u])

        def wait_idx(u):
            off = pl.multiple_of(base, 8)
            pltpu.make_async_copy(src_hbm.at[pl.ds(off, CH)], sidx.at[u, 0],
                                  isem[u]).wait()
            pltpu.make_async_copy(dst_hbm.at[pl.ds(off, CH)], didx.at[u, 0],
                                  isem[u]).wait()

        def fire_gather(u):
            pltpu.async_copy(x_hbm.at[sidx.at[u, 0]], rows.at[u], gsem[u])
            if with_scalar:
                pltpu.async_copy(dvec_hbm.at[sidx.at[u, 0]], svals.at[u, 0],
                                 gsem[u])

        def wait_gather(u):
            pltpu.make_async_copy(x_hbm.at[sidx.at[u, 0]], rows.at[u],
                                  gsem[u]).wait()
            if with_scalar:
                pltpu.make_async_copy(dvec_hbm.at[sidx.at[u, 0]],
                                      svals.at[u, 0], gsem[u]).wait()

        def fire_scatter(u):
            pltpu.async_copy(rows.at[u], yacc.at[didx.at[u, 0]], rsem[u],
                             add=True)
            if with_scalar:
                pltpu.async_copy(svals.at[u, 0], sacc.at[didx.at[u, 0]],
                                 rsem[u], add=True)

        def wait_scatter(u):
            pltpu.make_async_copy(rows.at[u], yacc.at[didx.at[u, 0]],
                                  rsem[u]).wait()
            if with_scalar:
                pltpu.make_async_copy(svals.at[u, 0],
                                      sacc.at[didx.at[u, 0]], rsem[u]).wait()

        def stage(g, par, do_next_gather, do_idx_load):
            s = par * GS
            s1 = (1 - par) * GS
            for b in range(GS):
                wait_gather(s + b)
                fire_scatter(s + b)
            if do_next_gather:
                for b in range(GS):
                    wait_idx(s1 + b)
                    fire_gather(s1 + b)
            for b in range(GS):
                wait_scatter(s + b)
                if do_idx_load:
                    fire_idx((g + 2) * GS + b, s + b)

        for b in range(GS):
            fire_idx(b, b)
        for b in range(GS):
            wait_idx(b)
            fire_gather(b)
        for b in range(GS):
            fire_idx(GS + b, GS + b)

        for i in range(ZR):
            for j in range(D // 16):
                zblk[i, pl.ds(j * 16, 16)] = jnp.zeros((16,), f32)
        for t in range(RPT // ZR):
            pltpu.sync_copy(zblk, yacc.at[pl.ds(r0 + t * ZR, ZR)])
        if with_scalar:
            for j in range(RPT // 16):
                zrow[pl.ds(j * 16, 16)] = jnp.zeros((16,), f32)
            pltpu.sync_copy(zrow, sacc.at[pl.ds(r0, RPT)])
        plsc.subcore_barrier()

        def body_g(k, carry):
            stage(2 * k, 0, True, True)
            stage(2 * k + 1, 1, True, True)
            return carry

        lax.fori_loop(0, (LASTG - 2) // 2, body_g, 0)
        stage(LASTG - 2, (LASTG - 2) % 2, True, True)
        stage(LASTG - 1, (LASTG - 1) % 2, True, False)
        stage(LASTG, LASTG % 2, False, False)
        plsc.subcore_barrier()
        pltpu.sync_copy(yacc.at[pl.ds(r0, RPT)], yout.at[cid, pl.ds(r0, RPT)])
        if with_scalar:
            pltpu.sync_copy(sacc.at[pl.ds(r0, RPT)],
                            sout.at[cid, pl.ds(r0, RPT)])

    return functools.partial(
        pl.kernel,
        out_type=tuple(out_type) if with_scalar else out_type[0],
        mesh=_mesh,
        scratch_types=scratch,
    )(body)


_agg_scalar = _make_agg(True)
_agg_plain = _make_agg(False)


def _unpack_col(d):
    pr = lax.broadcasted_iota(jnp.int32, (BT, BT // 128), 0) // 128
    pc = lax.broadcasted_iota(jnp.int32, (BT, BT // 128), 1)
    sel = (pr == pc).astype(f32)
    rows = jnp.dot(sel, d, preferred_element_type=f32)
    lr = lax.broadcasted_iota(jnp.int32, (BT, 128), 0) & 127
    lc = lax.broadcasted_iota(jnp.int32, (BT, 128), 1)
    msk = (lr == lc).astype(f32)
    return jnp.sum(rows * msk, axis=1, keepdims=True)


def _prep_body(c0_ref, c1_ref, emb_ref, t1_ref, disp_ref):
    deg = c0_ref[...] + c1_ref[...] + 1.0
    dis = lax.rsqrt(deg)
    disp_ref[...] = dis
    t1_ref[...] = _unpack_col(dis) * emb_ref[...]


def _mid_body(p0_ref, p1_ref, t1_ref, disp_ref, sv0_ref, sv1_ref,
              t3_ref, svecp_ref):
    dis = disp_ref[...]
    disr = _unpack_col(dis)
    t3_ref[...] = (disr * disr) * (p0_ref[...] + p1_ref[...] + t1_ref[...])
    svecp_ref[...] = dis * (sv0_ref[...] + sv1_ref[...] + dis)


def _final_body(q0_ref, q1_ref, t3_ref, disp_ref, svecp_ref, w1_ref, w2_ref,
                b1_ref, b2_ref, out_ref):
    disr = _unpack_col(disp_ref[...])
    svr = _unpack_col(svecp_ref[...])
    q = disr * (q0_ref[...] + q1_ref[...] + t3_ref[...])
    h = jnp.dot(q, w1_ref[...], preferred_element_type=f32) + svr * b1_ref[...]
    out_ref[...] = jnp.dot(h, w2_ref[...], preferred_element_type=f32) \
                   + b2_ref[...]


def _pack_spec():
    return pl.BlockSpec((BT // 128, 128), lambda i: (i, 0))


def _row_spec():
    return pl.BlockSpec((BT, D), lambda i: (i, 0))


def _full_spec(shape):
    return pl.BlockSpec(shape, lambda i: (0, 0))


NPK = NPAD // 128


def kernel(emb, edge_index, W1, b1, W2, b2):
    src1 = edge_index[0]
    dst1 = edge_index[1]
    dst3 = dst1.reshape(E // CH_D, 1, CH_D)
    embp = jnp.pad(emb, ((0, NPAD - N), (0, 0)))

    cnt = _deg_kernel(dst3)

    t1, disp = pl.pallas_call(
        _prep_body,
        grid=(GT,),
        in_specs=[_pack_spec(), _pack_spec(), _row_spec()],
        out_specs=[_row_spec(), _pack_spec()],
        out_shape=(jax.ShapeDtypeStruct((NPAD, D), f32),
                   jax.ShapeDtypeStruct((NPK, 128), f32)),
    )(cnt[0].reshape(NPK, 128), cnt[1].reshape(NPK, 128), embp)

    t2p, svp = _agg_scalar(t1, disp.reshape(NPAD), src1, dst1)

    t3, svecp = pl.pallas_call(
        _mid_body,
        grid=(GT,),
        in_specs=[_row_spec(), _row_spec(), _row_spec(), _pack_spec(),
                  _pack_spec(), _pack_spec()],
        out_specs=[_row_spec(), _pack_spec()],
        out_shape=(jax.ShapeDtypeStruct((NPAD, D), f32),
                   jax.ShapeDtypeStruct((NPK, 128), f32)),
    )(t2p[0], t2p[1], t1, disp,
      svp[0].reshape(NPK, 128), svp[1].reshape(NPK, 128))

    t4p = _agg_plain(t3, src1, dst1)

    out = pl.pallas_call(
        _final_body,
        grid=(GT,),
        in_specs=[_row_spec(), _row_spec(), _row_spec(), _pack_spec(),
                  _pack_spec(), _full_spec((D, 2 * D)),
                  _full_spec((2 * D, D)), _full_spec((1, 2 * D)),
                  _full_spec((1, D))],
        out_specs=_row_spec(),
        out_shape=jax.ShapeDtypeStruct((NPAD, D), f32),
    )(t4p[0], t4p[1], t3, disp, svecp, W1, W2, b1.reshape(1, -1),
      b2.reshape(1, -1))

    return out[:N]
